# Initial kernel scaffold; baseline (speedup 1.0000x reference)
import jax
import jax.numpy as jnp
from jax.experimental import pallas as pl


def kernel(x, edge_index, W_in, b_in, W_out, b_out, Ws):
    raise NotImplementedError("write your pallas kernel here")


# speedup vs baseline: 3.2154x; 3.2154x over previous
import functools

import numpy as np
import jax
import jax.numpy as jnp
from jax import lax
from jax.experimental import pallas as pl
from jax.experimental.pallas import tpu as pltpu
from jax.experimental.pallas import tpu_sc as plsc

ALPHA = 0.1
THETA = 0.5
NUM_LAYERS = 8
N_NODES = 10000
N_EDGES = 320000
D = 128

NC = 2
NS = 16
NW = NC * NS
CHUNK = 128
EDGES_PER_TILE = 10240
NCHUNK = EDGES_PER_TILE // CHUNK
E_PAD = NW * EDGES_PER_TILE
ACC_ROWS = 10240
ZROWS_PER_TILE = ACC_ROWS // NS
OUT_ROWS_PER_TILE = 624
TRASH_ROW = N_NODES

_mesh = plsc.VectorSubcoreMesh(core_axis_name="c", subcore_axis_name="s")


N_HALF = 2
CH_PER_HALF = NCHUNK // N_HALF
PAIRS = CH_PER_HALF // 2


def _agg_body(h_hbm, src_hbm, dst_hbm, out_hbm,
              src_v, dst_v, gbuf_a, gbuf_b, zbuf, acc, sem_a, sem_b):
    cid = lax.axis_index("c")
    sid = lax.axis_index("s")
    wid = sid * NC + cid

    zero16 = jnp.zeros((16,), jnp.float32)
    for r in range(16):
        for c8 in range(D // 16):
            zbuf[r, pl.ds(c8 * 16, 16)] = zero16

    def _zero(i, carry):
        pltpu.sync_copy(zbuf, acc.at[pl.ds(sid * ZROWS_PER_TILE + i * 16, 16)])
        return carry
    lax.fori_loop(0, ZROWS_PER_TILE // 16, _zero, 0)

    plsc.subcore_barrier()

    for half in range(N_HALF):
        pltpu.sync_copy(src_hbm.at[wid, pl.ds(half * CH_PER_HALF, CH_PER_HALF)],
                        src_v)
        pltpu.sync_copy(dst_hbm.at[wid, pl.ds(half * CH_PER_HALF, CH_PER_HALF)],
                        dst_v)
        pltpu.async_copy(h_hbm.at[src_v.at[0]], gbuf_a, sem_a)

        def _body(jj, carry):
            j0 = 2 * jj
            pltpu.async_copy(h_hbm.at[src_v.at[j0 + 1]], gbuf_b, sem_b)
            pltpu.make_async_copy(h_hbm.at[src_v.at[j0]], gbuf_a, sem_a).wait()
            pltpu.sync_copy(gbuf_a, acc.at[dst_v.at[j0]], add=True)

            @pl.when(jj + 1 < PAIRS)
            def _prefetch():
                pltpu.async_copy(h_hbm.at[src_v.at[j0 + 2]], gbuf_a, sem_a)

            pltpu.make_async_copy(h_hbm.at[src_v.at[j0 + 1]], gbuf_b, sem_b).wait()
            pltpu.sync_copy(gbuf_b, acc.at[dst_v.at[j0 + 1]], add=True)
            return carry
        lax.fori_loop(0, PAIRS, _body, 0)

    plsc.subcore_barrier()

    pltpu.sync_copy(
        acc.at[pl.ds(sid * OUT_ROWS_PER_TILE, OUT_ROWS_PER_TILE)],
        out_hbm.at[cid, pl.ds(sid * OUT_ROWS_PER_TILE, OUT_ROWS_PER_TILE)])

    tail = NS * OUT_ROWS_PER_TILE

    @pl.when(sid == NS - 1)
    def _tail():
        pltpu.sync_copy(
            acc.at[pl.ds(tail, N_NODES - tail)],
            out_hbm.at[cid, pl.ds(tail, N_NODES - tail)])


_agg = pl.kernel(
    _agg_body,
    out_type=jax.ShapeDtypeStruct((NC, N_NODES, D), jnp.float32),
    mesh=_mesh,
    scratch_types=[
        pltpu.VMEM((CH_PER_HALF, CHUNK), jnp.int32),
        pltpu.VMEM((CH_PER_HALF, CHUNK), jnp.int32),
        pltpu.VMEM((CHUNK, D), jnp.float32),
        pltpu.VMEM((CHUNK, D), jnp.float32),
        pltpu.VMEM((16, D), jnp.float32),
        pltpu.VMEM_SHARED((ACC_ROWS, D), jnp.float32),
        pltpu.SemaphoreType.DMA,
        pltpu.SemaphoreType.DMA,
    ],
)

_ROW_BLOCK = 2000


def _mm_bias_body(x_ref, w_ref, b_ref, o_ref, *, relu):
    acc = jnp.dot(x_ref[...], w_ref[...], preferred_element_type=jnp.float32,
                  precision=lax.Precision.HIGHEST)
    acc = acc + b_ref[...]
    if relu:
        acc = jnp.maximum(acc, 0.0)
    o_ref[...] = acc


def _mm_bias(x, w, b, relu):
    return pl.pallas_call(
        functools.partial(_mm_bias_body, relu=relu),
        grid=(N_NODES // _ROW_BLOCK,),
        in_specs=[pl.BlockSpec((_ROW_BLOCK, D), lambda i: (i, 0)),
                  pl.BlockSpec((D, D), lambda i: (0, 0)),
                  pl.BlockSpec((1, D), lambda i: (0, 0))],
        out_specs=pl.BlockSpec((_ROW_BLOCK, D), lambda i: (i, 0)),
        out_shape=jax.ShapeDtypeStruct((N_NODES, D), jnp.float32),
    )(x, w, b.reshape(1, D))


def _layer_body(p_ref, x0_ref, w_ref, o_ref, *, beta):
    hx = (1.0 - ALPHA) * (p_ref[0] + p_ref[1]) + ALPHA * x0_ref[...]
    mm = jnp.dot(hx, w_ref[...], preferred_element_type=jnp.float32,
                 precision=lax.Precision.HIGHEST)
    o_ref[...] = jnp.maximum((1.0 - beta) * hx + beta * mm, 0.0)


def _layer_update(partials, x0, w, beta):
    return pl.pallas_call(
        functools.partial(_layer_body, beta=beta),
        grid=(N_NODES // _ROW_BLOCK,),
        in_specs=[pl.BlockSpec((NC, _ROW_BLOCK, D), lambda i: (0, i, 0)),
                  pl.BlockSpec((_ROW_BLOCK, D), lambda i: (i, 0)),
                  pl.BlockSpec((D, D), lambda i: (0, 0))],
        out_specs=pl.BlockSpec((_ROW_BLOCK, D), lambda i: (i, 0)),
        out_shape=jax.ShapeDtypeStruct((N_NODES, D), jnp.float32),
    )(partials, x0, w)


def kernel(x, edge_index, W_in, b_in, W_out, b_out, Ws):
    src = edge_index[0].astype(jnp.int32)
    dst = edge_index[1].astype(jnp.int32)
    pad = E_PAD - N_EDGES
    src_p = jnp.concatenate([src, jnp.zeros((pad,), jnp.int32)])
    dst_p = jnp.concatenate([dst, jnp.full((pad,), TRASH_ROW, jnp.int32)])
    src_p = src_p.reshape(NW, NCHUNK, CHUNK)
    dst_p = dst_p.reshape(NW, NCHUNK, CHUNK)

    h = _mm_bias(x, W_in, b_in, relu=True)
    x0 = h
    for layer in range(NUM_LAYERS):
        beta = float(np.log(THETA / (layer + 1) + 1.0))
        partials = _agg(h, src_p, dst_p)
        h = _layer_update(partials, x0, w=Ws[layer], beta=beta)
    return _mm_bias(h, W_out, b_out, relu=False)


# speedup vs baseline: 3.9037x; 1.2141x over previous
import functools

import numpy as np
import jax
import jax.numpy as jnp
from jax import lax
from jax.experimental import pallas as pl
from jax.experimental.pallas import tpu as pltpu
from jax.experimental.pallas import tpu_sc as plsc

ALPHA = 0.1
THETA = 0.5
NUM_LAYERS = 8
N_NODES = 10000
N_EDGES = 320000
D = 128

NC = 2
NS = 16
NW = NC * NS
CHUNK = 64
NBUF = 4
EDGES_PER_TILE = 10240
NCHUNK = EDGES_PER_TILE // CHUNK
E_PAD = NW * EDGES_PER_TILE
ACC_ROWS = 10240
ZROWS_PER_TILE = ACC_ROWS // NS
OUT_ROWS_PER_TILE = 624
TRASH_ROW = N_NODES

_mesh = plsc.VectorSubcoreMesh(core_axis_name="c", subcore_axis_name="s")


N_HALF = 4
CH_PER_HALF = NCHUNK // N_HALF


def _agg_body(h_hbm, src_hbm, dst_hbm, out_hbm,
              src_v, dst_v, zbuf, acc, *gbufs_sems):
    gbufs = gbufs_sems[:NBUF]
    sems = gbufs_sems[NBUF:]
    cid = lax.axis_index("c")
    sid = lax.axis_index("s")
    wid = sid * NC + cid

    zero16 = jnp.zeros((16,), jnp.float32)
    for r in range(16):
        for c8 in range(D // 16):
            zbuf[r, pl.ds(c8 * 16, 16)] = zero16

    def _zero(i, carry):
        pltpu.sync_copy(zbuf, acc.at[pl.ds(sid * ZROWS_PER_TILE + i * 16, 16)])
        return carry
    lax.fori_loop(0, ZROWS_PER_TILE // 16, _zero, 0)

    plsc.subcore_barrier()

    for half in range(N_HALF):
        pltpu.sync_copy(src_hbm.at[wid, pl.ds(half * CH_PER_HALF, CH_PER_HALF)],
                        src_v)
        pltpu.sync_copy(dst_hbm.at[wid, pl.ds(half * CH_PER_HALF, CH_PER_HALF)],
                        dst_v)
        for b in range(NBUF):
            pltpu.async_copy(h_hbm.at[src_v.at[b]], gbufs[b], sems[b])

        def _body(g, carry):
            j0 = g * NBUF
            for b in range(NBUF):
                j = j0 + b
                pltpu.make_async_copy(h_hbm.at[src_v.at[j]], gbufs[b],
                                      sems[b]).wait()
                pltpu.sync_copy(gbufs[b], acc.at[dst_v.at[j]], add=True)

                @pl.when(j + NBUF < CH_PER_HALF)
                def _prefetch(j=j, b=b):
                    pltpu.async_copy(h_hbm.at[src_v.at[j + NBUF]], gbufs[b],
                                     sems[b])
            return carry
        lax.fori_loop(0, CH_PER_HALF // NBUF, _body, 0)

    plsc.subcore_barrier()

    pltpu.sync_copy(
        acc.at[pl.ds(sid * OUT_ROWS_PER_TILE, OUT_ROWS_PER_TILE)],
        out_hbm.at[cid, pl.ds(sid * OUT_ROWS_PER_TILE, OUT_ROWS_PER_TILE)])

    tail = NS * OUT_ROWS_PER_TILE

    @pl.when(sid == NS - 1)
    def _tail():
        pltpu.sync_copy(
            acc.at[pl.ds(tail, N_NODES - tail)],
            out_hbm.at[cid, pl.ds(tail, N_NODES - tail)])


_agg = pl.kernel(
    _agg_body,
    out_type=jax.ShapeDtypeStruct((NC, N_NODES, D), jnp.float32),
    mesh=_mesh,
    scratch_types=(
        [pltpu.VMEM((CH_PER_HALF, CHUNK), jnp.int32),
         pltpu.VMEM((CH_PER_HALF, CHUNK), jnp.int32),
         pltpu.VMEM((16, D), jnp.float32),
         pltpu.VMEM_SHARED((ACC_ROWS, D), jnp.float32)]
        + [pltpu.VMEM((CHUNK, D), jnp.float32) for _ in range(NBUF)]
        + [pltpu.SemaphoreType.DMA for _ in range(NBUF)]
    ),
)

_ROW_BLOCK = 2000


def _mm_bias_body(x_ref, w_ref, b_ref, o_ref, *, relu):
    acc = jnp.dot(x_ref[...], w_ref[...], preferred_element_type=jnp.float32,
                  precision=lax.Precision.HIGHEST)
    acc = acc + b_ref[...]
    if relu:
        acc = jnp.maximum(acc, 0.0)
    o_ref[...] = acc


def _mm_bias(x, w, b, relu):
    return pl.pallas_call(
        functools.partial(_mm_bias_body, relu=relu),
        grid=(N_NODES // _ROW_BLOCK,),
        in_specs=[pl.BlockSpec((_ROW_BLOCK, D), lambda i: (i, 0)),
                  pl.BlockSpec((D, D), lambda i: (0, 0)),
                  pl.BlockSpec((1, D), lambda i: (0, 0))],
        out_specs=pl.BlockSpec((_ROW_BLOCK, D), lambda i: (i, 0)),
        out_shape=jax.ShapeDtypeStruct((N_NODES, D), jnp.float32),
    )(x, w, b.reshape(1, D))


def _layer_body(p_ref, x0_ref, w_ref, o_ref, *, beta):
    hx = (1.0 - ALPHA) * (p_ref[0] + p_ref[1]) + ALPHA * x0_ref[...]
    mm = jnp.dot(hx, w_ref[...], preferred_element_type=jnp.float32,
                 precision=lax.Precision.HIGHEST)
    o_ref[...] = jnp.maximum((1.0 - beta) * hx + beta * mm, 0.0)


def _layer_update(partials, x0, w, beta):
    return pl.pallas_call(
        functools.partial(_layer_body, beta=beta),
        grid=(N_NODES // _ROW_BLOCK,),
        in_specs=[pl.BlockSpec((NC, _ROW_BLOCK, D), lambda i: (0, i, 0)),
                  pl.BlockSpec((_ROW_BLOCK, D), lambda i: (i, 0)),
                  pl.BlockSpec((D, D), lambda i: (0, 0))],
        out_specs=pl.BlockSpec((_ROW_BLOCK, D), lambda i: (i, 0)),
        out_shape=jax.ShapeDtypeStruct((N_NODES, D), jnp.float32),
    )(partials, x0, w)


def kernel(x, edge_index, W_in, b_in, W_out, b_out, Ws):
    src = edge_index[0].astype(jnp.int32)
    dst = edge_index[1].astype(jnp.int32)
    pad = E_PAD - N_EDGES
    src_p = jnp.concatenate([src, jnp.zeros((pad,), jnp.int32)])
    dst_p = jnp.concatenate([dst, jnp.full((pad,), TRASH_ROW, jnp.int32)])
    src_p = src_p.reshape(NW, NCHUNK, CHUNK)
    dst_p = dst_p.reshape(NW, NCHUNK, CHUNK)

    h = _mm_bias(x, W_in, b_in, relu=True)
    x0 = h
    for layer in range(NUM_LAYERS):
        beta = float(np.log(THETA / (layer + 1) + 1.0))
        partials = _agg(h, src_p, dst_p)
        h = _layer_update(partials, x0, w=Ws[layer], beta=beta)
    return _mm_bias(h, W_out, b_out, relu=False)
